# Initial kernel scaffold; baseline (speedup 1.0000x reference)
import jax
import jax.numpy as jnp
from jax.experimental import pallas as pl


def kernel(boxes, scores, max_output_size):
    raise NotImplementedError("write your pallas kernel here")


# speedup vs baseline: 20.5958x; 20.5958x over previous
import functools

import jax
import jax.numpy as jnp
from jax import lax
from jax.experimental import pallas as pl
from jax.experimental.pallas import tpu as pltpu
from jax.experimental.pallas import tpu_sc as plsc

N_PAD = 5120
PER_W = N_PAD // 16
VREGS = PER_W // 16
MAX_OUT = 100


def _splat(x):
    return jnp.full((16,), x)


def _nms_body(y1h, x1h, y2h, x2h, sh, outh,
              y1v, x1v, y2v, x2v, sv, areav,
              stage, table_sh, tablev, outv):
    cid = lax.axis_index("c")
    wid = lax.axis_index("s")
    base = wid * PER_W
    iota = lax.iota(jnp.int32, 16)

    pltpu.sync_copy(y1h.at[pl.ds(base, PER_W)], y1v)
    pltpu.sync_copy(x1h.at[pl.ds(base, PER_W)], x1v)
    pltpu.sync_copy(y2h.at[pl.ds(base, PER_W)], y2v)
    pltpu.sync_copy(x2h.at[pl.ds(base, PER_W)], x2v)
    pltpu.sync_copy(sh.at[pl.ds(base, PER_W)], sv)

    for j in range(VREGS):
        sl = pl.ds(j * 16, 16)
        areav[sl] = (y2v[sl] - y1v[sl]) * (x2v[sl] - x1v[sl])

    zero = jnp.zeros((16,), jnp.float32)

    def body(t, carry):
        py1, px1, py2, px2, pa = carry

        best = jnp.full((16,), -2.0)
        bidx = jnp.zeros((16,), jnp.int32)
        for j in range(VREGS):
            sl = pl.ds(j * 16, 16)
            iy1 = jnp.maximum(py1, y1v[sl])
            ix1 = jnp.maximum(px1, x1v[sl])
            iy2 = jnp.minimum(py2, y2v[sl])
            ix2 = jnp.minimum(px2, x2v[sl])
            inter = jnp.maximum(iy2 - iy1, 0.0) * jnp.maximum(ix2 - ix1, 0.0)
            union = pa + areav[sl] - inter
            s = jnp.where(inter + inter > union, -1.0, sv[sl])
            sv[sl] = s
            gt = s > best
            best = jnp.where(gt, s, best)
            bidx = jnp.where(gt, base + (j * 16) + iota, bidx)

        lmax = jnp.max(best)
        lidx = jnp.min(jnp.where(best == lmax, bidx, jnp.int32(1 << 30)))

        li = _splat(lidx - base)
        wy1 = plsc.load_gather(y1v, [li])
        wx1 = plsc.load_gather(x1v, [li])
        wy2 = plsc.load_gather(y2v, [li])
        wx2 = plsc.load_gather(x2v, [li])

        gsp = _splat(lmax)
        lsp = _splat(lidx.astype(jnp.float32))
        row = jnp.where(iota == 0, gsp,
              jnp.where(iota == 1, lsp,
              jnp.where(iota == 2, wy1,
              jnp.where(iota == 3, wx1,
              jnp.where(iota == 4, wy2, wx2)))))
        stage[...] = row
        pltpu.sync_copy(stage, table_sh.at[pl.ds(wid * 16, 16)])
        plsc.subcore_barrier()
        pltpu.sync_copy(table_sh, tablev)
        plsc.subcore_barrier()

        col = iota * 16
        vals = plsc.load_gather(tablev, [col])
        gidx = plsc.load_gather(tablev, [col + 1])
        gmax = jnp.max(vals)
        widf = jnp.min(jnp.where(vals == gmax, gidx, jnp.float32(1e9)))
        rowm = jnp.logical_and(vals == gmax, gidx == widf)
        wrow = jnp.min(jnp.where(rowm, iota, jnp.int32(999)))
        rb = wrow * 16
        npy1 = plsc.load_gather(tablev, [_splat(rb + 2)])
        npx1 = plsc.load_gather(tablev, [_splat(rb + 3)])
        npy2 = plsc.load_gather(tablev, [_splat(rb + 4)])
        npx2 = plsc.load_gather(tablev, [_splat(rb + 5)])
        npa = (npy2 - npy1) * (npx2 - npx1)

        has = gmax >= 0.0
        hasf = _splat(jnp.where(has, 1.0, 0.0).astype(jnp.float32))

        @pl.when(jnp.logical_and(cid == 0, wid == 0))
        def _():
            v = jnp.where(iota == 0, npy1,
                jnp.where(iota == 1, npx1,
                jnp.where(iota == 2, npy2, npx2))) * hasf
            plsc.store_scatter(outv, [t * 4 + iota], v, mask=iota < 4)

        widi = widf.astype(jnp.int32)
        own = jnp.logical_and(widi >= base, widi < base + PER_W)
        @pl.when(jnp.logical_and(own, has))
        def _():
            plsc.store_scatter(sv, [_splat(widi - base)], jnp.full((16,), -1.0),
                               mask=iota == 0)

        return (npy1, npx1, npy2, npx2, npa)

    lax.fori_loop(0, MAX_OUT, body, (zero, zero, zero, zero, zero),
                  unroll=False)

    @pl.when(jnp.logical_and(cid == 0, wid == 0))
    def _():
        pltpu.sync_copy(outv.at[pl.ds(0, MAX_OUT * 4)], outh)


@jax.jit
def _nms(y1, x1, y2, x2, s):
    mesh = plsc.VectorSubcoreMesh(core_axis_name="c", subcore_axis_name="s")
    f = functools.partial(
        pl.kernel,
        mesh=mesh,
        compiler_params=pltpu.CompilerParams(needs_layout_passes=False),
        out_type=jax.ShapeDtypeStruct((MAX_OUT * 4,), jnp.float32),
        scratch_types=[
            pltpu.VMEM((PER_W,), jnp.float32),
            pltpu.VMEM((PER_W,), jnp.float32),
            pltpu.VMEM((PER_W,), jnp.float32),
            pltpu.VMEM((PER_W,), jnp.float32),
            pltpu.VMEM((PER_W,), jnp.float32),
            pltpu.VMEM((PER_W,), jnp.float32),
            pltpu.VMEM((16,), jnp.float32),
            pltpu.VMEM_SHARED((256,), jnp.float32),
            pltpu.VMEM((256,), jnp.float32),
            pltpu.VMEM((MAX_OUT * 4 + 16,), jnp.float32),
        ],
    )(_nms_body)
    return f(y1, x1, y2, x2, s)


def kernel(boxes, scores, max_output_size):
    n = boxes.shape[0]
    pad = N_PAD - n
    y1 = jnp.pad(boxes[:, 0], (0, pad))
    x1 = jnp.pad(boxes[:, 1], (0, pad))
    y2 = jnp.pad(boxes[:, 2], (0, pad))
    x2 = jnp.pad(boxes[:, 3], (0, pad))
    s = jnp.pad(scores, (0, pad), constant_values=-1.0)
    out = _nms(y1, x1, y2, x2, s).reshape(MAX_OUT, 4)
    keep = (lax.iota(jnp.int32, MAX_OUT) < max_output_size)[:, None]
    return jnp.where(keep, out, 0.0)


# speedup vs baseline: 21.6417x; 1.0508x over previous
import functools

import jax
import jax.numpy as jnp
from jax import lax
from jax.experimental import pallas as pl
from jax.experimental.pallas import tpu as pltpu
from jax.experimental.pallas import tpu_sc as plsc

N_PAD = 5120
PER_W = N_PAD // 16
VREGS = PER_W // 16
MAX_OUT = 100
ROW = 8


def _splat(x):
    return jnp.full((16,), x)


def _nms_body(y1h, x1h, y2h, x2h, sh, outh,
              y1v, x1v, y2v, x2v, sv, areav,
              stage, table_sh, tablev, outv):
    cid = lax.axis_index("c")
    wid = lax.axis_index("s")
    base = wid * PER_W
    iota = lax.iota(jnp.int32, 16)

    pltpu.sync_copy(y1h.at[pl.ds(base, PER_W)], y1v)
    pltpu.sync_copy(x1h.at[pl.ds(base, PER_W)], x1v)
    pltpu.sync_copy(y2h.at[pl.ds(base, PER_W)], y2v)
    pltpu.sync_copy(x2h.at[pl.ds(base, PER_W)], x2v)
    pltpu.sync_copy(sh.at[pl.ds(base, PER_W)], sv)

    scores0 = []
    for j in range(VREGS):
        sl = pl.ds(j * 16, 16)
        areav[sl] = (y2v[sl] - y1v[sl]) * (x2v[sl] - x1v[sl])
        scores0.append(sv[sl])

    zero = jnp.zeros((16,), jnp.float32)

    def body(t, carry):
        py1, px1, py2, px2, pa = carry[:5]
        scores = carry[5:]

        best = jnp.full((16,), -2.0)
        bidx = jnp.zeros((16,), jnp.int32)
        idxv = base + iota
        new_scores = []
        for j in range(VREGS):
            sl = pl.ds(j * 16, 16)
            iy1 = jnp.maximum(py1, y1v[sl])
            ix1 = jnp.maximum(px1, x1v[sl])
            iy2 = jnp.minimum(py2, y2v[sl])
            ix2 = jnp.minimum(px2, x2v[sl])
            inter = jnp.maximum(iy2 - iy1, 0.0) * jnp.maximum(ix2 - ix1, 0.0)
            union = pa + areav[sl] - inter
            s = jnp.where(inter + inter > union, -1.0, scores[j])
            new_scores.append(s)
            gt = s > best
            best = jnp.where(gt, s, best)
            bidx = jnp.where(gt, idxv, bidx)
            idxv = idxv + 16

        lmax = jnp.max(best)
        lidx = jnp.min(jnp.where(best == lmax, bidx, jnp.int32(1 << 30)))

        li = _splat(lidx - base)
        wy1 = plsc.load_gather(y1v, [li])
        wx1 = plsc.load_gather(x1v, [li])
        wy2 = plsc.load_gather(y2v, [li])
        wx2 = plsc.load_gather(x2v, [li])

        gsp = _splat(lmax)
        lsp = _splat(lidx.astype(jnp.float32))
        row = jnp.where(iota == 0, gsp,
              jnp.where(iota == 1, lsp,
              jnp.where(iota == 2, wy1,
              jnp.where(iota == 3, wx1,
              jnp.where(iota == 4, wy2, wx2)))))
        stage[...] = row
        off = (t & 1) * (16 * ROW)
        pltpu.sync_copy(stage.at[pl.ds(0, ROW)],
                        table_sh.at[pl.ds(off + wid * ROW, ROW)])
        plsc.subcore_barrier()
        pltpu.sync_copy(table_sh.at[pl.ds(off, 16 * ROW)], tablev)

        col = iota * ROW
        vals = plsc.load_gather(tablev, [col])
        gidx = plsc.load_gather(tablev, [col + 1])
        gmax = jnp.max(vals)
        widf = jnp.min(jnp.where(vals == gmax, gidx, jnp.float32(1e9)))
        rowm = jnp.logical_and(vals == gmax, gidx == widf)
        wrow = jnp.min(jnp.where(rowm, iota, jnp.int32(999)))
        rb = wrow * ROW
        npy1 = plsc.load_gather(tablev, [_splat(rb + 2)])
        npx1 = plsc.load_gather(tablev, [_splat(rb + 3)])
        npy2 = plsc.load_gather(tablev, [_splat(rb + 4)])
        npx2 = plsc.load_gather(tablev, [_splat(rb + 5)])
        npa = (npy2 - npy1) * (npx2 - npx1)

        has = gmax >= 0.0
        hasf = _splat(jnp.where(has, 1.0, 0.0).astype(jnp.float32))

        @pl.when(jnp.logical_and(cid == 0, wid == 0))
        def _():
            v = jnp.where(iota == 0, npy1,
                jnp.where(iota == 1, npx1,
                jnp.where(iota == 2, npy2, npx2))) * hasf
            plsc.store_scatter(outv, [t * 4 + iota], v, mask=iota < 4)

        return (npy1, npx1, npy2, npx2, npa, *new_scores)

    lax.fori_loop(0, MAX_OUT, body, (zero, zero, zero, zero, zero, *scores0),
                  unroll=False)

    @pl.when(jnp.logical_and(cid == 0, wid == 0))
    def _():
        pltpu.sync_copy(outv.at[pl.ds(0, MAX_OUT * 4)], outh)


@jax.jit
def _nms(y1, x1, y2, x2, s):
    mesh = plsc.VectorSubcoreMesh(core_axis_name="c", subcore_axis_name="s")
    f = functools.partial(
        pl.kernel,
        mesh=mesh,
        compiler_params=pltpu.CompilerParams(needs_layout_passes=False),
        out_type=jax.ShapeDtypeStruct((MAX_OUT * 4,), jnp.float32),
        scratch_types=[
            pltpu.VMEM((PER_W,), jnp.float32),
            pltpu.VMEM((PER_W,), jnp.float32),
            pltpu.VMEM((PER_W,), jnp.float32),
            pltpu.VMEM((PER_W,), jnp.float32),
            pltpu.VMEM((PER_W,), jnp.float32),
            pltpu.VMEM((PER_W,), jnp.float32),
            pltpu.VMEM((16,), jnp.float32),
            pltpu.VMEM_SHARED((2 * 16 * ROW,), jnp.float32),
            pltpu.VMEM((16 * ROW,), jnp.float32),
            pltpu.VMEM((MAX_OUT * 4 + 16,), jnp.float32),
        ],
    )(_nms_body)
    return f(y1, x1, y2, x2, s)


def kernel(boxes, scores, max_output_size):
    n = boxes.shape[0]
    pad = N_PAD - n
    y1 = jnp.pad(boxes[:, 0], (0, pad))
    x1 = jnp.pad(boxes[:, 1], (0, pad))
    y2 = jnp.pad(boxes[:, 2], (0, pad))
    x2 = jnp.pad(boxes[:, 3], (0, pad))
    s = jnp.pad(scores, (0, pad), constant_values=-1.0)
    out = _nms(y1, x1, y2, x2, s).reshape(MAX_OUT, 4)
    keep = (lax.iota(jnp.int32, MAX_OUT) < max_output_size)[:, None]
    return jnp.where(keep, out, 0.0)
